# baseline (speedup 1.0000x reference)
import jax
import jax.numpy as jnp
from jax.experimental import pallas as pl
from jax.experimental.pallas import tpu as pltpu

_S = 0.5
_EPS = 1e-6
_LANE = 128
_BB = 512
_BT = 4096
_CB = _BT // _LANE
_LOG2E = 1.4426950408889634


def _pcen_body(scal_ref, w_ref, halo_ref, x_ref, o_ref):
    t = pl.program_id(1)
    r = scal_ref[0]
    delta = scal_ref[1]
    dr = scal_ref[2]
    c_alpha = scal_ref[3]
    c_r = scal_ref[4]
    r_is_half = r == jnp.float32(0.5)
    halo = jnp.where(t == 0, 0.0, halo_ref[...])
    xw = jnp.concatenate([halo, x_ref[...]], axis=1)
    w = w_ref[...]
    def chunk_q(c):
        win = xw[:, c * _LANE:(c + 2) * _LANE]
        m = jnp.dot(win, w, preferred_element_type=jnp.float32,
                    precision=jax.lax.Precision.DEFAULT)
        xc = xw[:, (c + 1) * _LANE:(c + 2) * _LANE]
        p = jnp.exp2(c_alpha * jnp.log(_EPS + m))
        return xc * p + delta

    @pl.when(r_is_half)
    def _():
        for c in range(_CB):
            q = chunk_q(c)
            o_ref[:, c * _LANE:(c + 1) * _LANE] = q * jax.lax.rsqrt(q) - dr

    @pl.when(jnp.logical_not(r_is_half))
    def _():
        for c in range(_CB):
            q = chunk_q(c)
            o_ref[:, c * _LANE:(c + 1) * _LANE] = (
                jnp.exp2(c_r * jnp.log(q)) - dr)


def _pcen_call(x, alpha, r, delta):
    nb, T = x.shape
    af = jnp.asarray(alpha, jnp.float32)
    rf = jnp.asarray(r, jnp.float32)
    df = jnp.asarray(delta, jnp.float32)
    scal = jnp.stack([rf, df, df ** rf, -af * _LOG2E, rf * _LOG2E])
    wi = jax.lax.broadcasted_iota(jnp.float32, (2 * _LANE, _LANE), 0)
    wj = jax.lax.broadcasted_iota(jnp.float32, (2 * _LANE, _LANE), 1)
    d = wj + float(_LANE) - wi
    w = jnp.where(d >= 0, _S * jnp.exp2(-d), 0.0).astype(jnp.float32)
    return pl.pallas_call(
        _pcen_body,
        out_shape=jax.ShapeDtypeStruct((nb, T), jnp.float32),
        grid=(nb // _BB, T // _BT),
        in_specs=[
            pl.BlockSpec(memory_space=pltpu.SMEM),
            pl.BlockSpec((2 * _LANE, _LANE), lambda i, t: (0, 0)),
            pl.BlockSpec((_BB, _LANE),
                         lambda i, t: (i, jnp.maximum(t * _CB - 1, 0))),
            pl.BlockSpec((_BB, _BT), lambda i, t: (i, t)),
        ],
        out_specs=pl.BlockSpec((_BB, _BT), lambda i, t: (i, t)),
        compiler_params=pltpu.CompilerParams(
            dimension_semantics=("parallel", "arbitrary"),
        ),
        name="pcen",
    )(scal, w, x, x)


def kernel(x, alpha, r, delta):
    nb = x.shape[0]
    devs = jax.devices()
    nd = len(devs)
    if nd > 1 and nb % (nd * _BB) == 0:
        mesh = jax.sharding.Mesh(devs, ("d",))
        f = jax.experimental.shard_map.shard_map(
            _pcen_call, mesh=mesh,
            in_specs=(jax.sharding.PartitionSpec("d", None),
                      jax.sharding.PartitionSpec(),
                      jax.sharding.PartitionSpec(),
                      jax.sharding.PartitionSpec()),
            out_specs=jax.sharding.PartitionSpec("d", None),
            check_rep=False)
        return f(x, alpha, r, delta)
    return _pcen_call(x, alpha, r, delta)


# speedup vs baseline: 6.0390x; 6.0390x over previous
import jax
import jax.numpy as jnp
from jax.experimental import pallas as pl
from jax.experimental.pallas import tpu as pltpu

_S = 0.5
_EPS = 1e-6
_LANE = 128
_BB = 64
_BT = 32768
_CB = _BT // _LANE
_LOG2E = 1.4426950408889634


def _pcen_body(scal_ref, w_ref, halo_ref, x_ref, o_ref):
    t = pl.program_id(1)
    r = scal_ref[0]
    delta = scal_ref[1]
    dr = scal_ref[2]
    c_alpha = scal_ref[3]
    c_r = scal_ref[4]
    r_is_half = r == jnp.float32(0.5)
    halo = jnp.where(t == 0, 0.0, halo_ref[...])
    xw = jnp.concatenate([halo, x_ref[...]], axis=1)
    w = w_ref[...]
    def chunk_q(c):
        win = xw[:, c * _LANE:(c + 2) * _LANE]
        m = jnp.dot(win, w, preferred_element_type=jnp.float32,
                    precision=jax.lax.Precision.DEFAULT)
        xc = xw[:, (c + 1) * _LANE:(c + 2) * _LANE]
        p = jnp.exp2(c_alpha * jnp.log(_EPS + m))
        return xc * p + delta

    @pl.when(r_is_half)
    def _():
        for c in range(_CB):
            q = chunk_q(c)
            o_ref[:, c * _LANE:(c + 1) * _LANE] = q * jax.lax.rsqrt(q) - dr

    @pl.when(jnp.logical_not(r_is_half))
    def _():
        for c in range(_CB):
            q = chunk_q(c)
            o_ref[:, c * _LANE:(c + 1) * _LANE] = (
                jnp.exp2(c_r * jnp.log(q)) - dr)


def kernel(x, alpha, r, delta):
    nb, T = x.shape
    af = jnp.asarray(alpha, jnp.float32)
    rf = jnp.asarray(r, jnp.float32)
    df = jnp.asarray(delta, jnp.float32)
    scal = jnp.stack([rf, df, df ** rf, -af * _LOG2E, rf * _LOG2E])
    wi = jax.lax.broadcasted_iota(jnp.float32, (2 * _LANE, _LANE), 0)
    wj = jax.lax.broadcasted_iota(jnp.float32, (2 * _LANE, _LANE), 1)
    d = wj + float(_LANE) - wi
    w = jnp.where(d >= 0, _S * jnp.exp2(-d), 0.0).astype(jnp.float32)
    return pl.pallas_call(
        _pcen_body,
        out_shape=jax.ShapeDtypeStruct((nb, T), jnp.float32),
        grid=(nb // _BB, T // _BT),
        in_specs=[
            pl.BlockSpec(memory_space=pltpu.SMEM),
            pl.BlockSpec((2 * _LANE, _LANE), lambda i, t: (0, 0)),
            pl.BlockSpec((_BB, _LANE),
                         lambda i, t: (i, jnp.maximum(t * _CB - 1, 0))),
            pl.BlockSpec((_BB, _BT), lambda i, t: (i, t)),
        ],
        out_specs=pl.BlockSpec((_BB, _BT), lambda i, t: (i, t)),
        compiler_params=pltpu.CompilerParams(
            dimension_semantics=("parallel", "arbitrary"),
        ),
        name="pcen",
    )(scal, w, x, x)


# speedup vs baseline: 6.1234x; 1.0140x over previous
import jax
import jax.numpy as jnp
import numpy as np
from jax.experimental import pallas as pl
from jax.experimental.pallas import tpu as pltpu

_S = 0.5
_EPS = 1e-6
_LANE = 128
_BB = 64
_BT = 32768
_CB = _BT // _LANE
_LOG2E = 1.4426950408889634

_Wd = (np.arange(_LANE)[None, :] + _LANE) - np.arange(2 * _LANE)[:, None]
_W = np.where(_Wd >= 0, _S * np.exp2(-_Wd.astype(np.float64)), 0.0).astype(
    np.float32)


def _pcen_body(scal_ref, w_ref, halo_ref, x_ref, o_ref):
    t = pl.program_id(1)
    r = scal_ref[0]
    delta = scal_ref[1]
    dr = scal_ref[2]
    c_alpha = scal_ref[3]
    c_r = scal_ref[4]
    r_is_half = r == jnp.float32(0.5)
    halo = jnp.where(t == 0, 0.0, halo_ref[...])
    xw = jnp.concatenate([halo, x_ref[...]], axis=1)
    w = w_ref[...]
    def chunk_q(c):
        win = xw[:, c * _LANE:(c + 2) * _LANE]
        m = jnp.dot(win, w, preferred_element_type=jnp.float32,
                    precision=jax.lax.Precision.DEFAULT)
        xc = xw[:, (c + 1) * _LANE:(c + 2) * _LANE]
        p = jnp.exp2(c_alpha * jnp.log(_EPS + m))
        return xc * p + delta

    @pl.when(r_is_half)
    def _():
        for c in range(_CB):
            q = chunk_q(c)
            o_ref[:, c * _LANE:(c + 1) * _LANE] = q * jax.lax.rsqrt(q) - dr

    @pl.when(jnp.logical_not(r_is_half))
    def _():
        for c in range(_CB):
            q = chunk_q(c)
            o_ref[:, c * _LANE:(c + 1) * _LANE] = (
                jnp.exp2(c_r * jnp.log(q)) - dr)


def kernel(x, alpha, r, delta):
    nb, T = x.shape
    af = jnp.asarray(alpha, jnp.float32)
    rf = jnp.asarray(r, jnp.float32)
    df = jnp.asarray(delta, jnp.float32)
    scal = jnp.stack([rf, df, df ** rf, -af * _LOG2E, rf * _LOG2E])
    w = jnp.asarray(_W)
    return pl.pallas_call(
        _pcen_body,
        out_shape=jax.ShapeDtypeStruct((nb, T), jnp.float32),
        grid=(nb // _BB, T // _BT),
        in_specs=[
            pl.BlockSpec(memory_space=pltpu.SMEM),
            pl.BlockSpec((2 * _LANE, _LANE), lambda i, t: (0, 0)),
            pl.BlockSpec((_BB, _LANE),
                         lambda i, t: (i, jnp.maximum(t * _CB - 1, 0))),
            pl.BlockSpec((_BB, _BT), lambda i, t: (i, t)),
        ],
        out_specs=pl.BlockSpec((_BB, _BT), lambda i, t: (i, t)),
        compiler_params=pltpu.CompilerParams(
            dimension_semantics=("parallel", "arbitrary"),
        ),
        name="pcen",
    )(scal, w, x, x)


# speedup vs baseline: 6.2534x; 1.0212x over previous
import jax
import jax.numpy as jnp
import numpy as np
from jax.experimental import pallas as pl
from jax.experimental.pallas import tpu as pltpu

_S = 0.5
_EPS = 1e-6
_LANE = 128
_BB = 64
_LOG2E = 1.4426950408889634

_Wd = (np.arange(_LANE)[None, :] + _LANE) - np.arange(2 * _LANE)[:, None]
_W = np.where(_Wd >= 0, _S * np.exp2(-_Wd.astype(np.float64)), 0.0).astype(
    np.float32)


def _pcen_body(scal_ref, w_ref, x_ref, o_ref):
    r = scal_ref[0]
    delta = scal_ref[1]
    dr = scal_ref[2]
    c_alpha = scal_ref[3]
    c_r = scal_ref[4]
    r_is_half = r == jnp.float32(0.5)
    n_chunks = x_ref.shape[1] // _LANE
    w = w_ref[...]

    def chunk_q(c):
        lo = (c - 1) * _LANE if c else 0
        win = x_ref[:, lo:(c + 1) * _LANE]
        wc = w[_LANE:, :] if c == 0 else w
        m = jnp.dot(win, wc, preferred_element_type=jnp.float32,
                    precision=jax.lax.Precision.DEFAULT)
        xc = x_ref[:, c * _LANE:(c + 1) * _LANE]
        p = jnp.exp2(c_alpha * jnp.log(_EPS + m))
        return xc * p + delta

    @pl.when(r_is_half)
    def _():
        for c in range(n_chunks):
            q = chunk_q(c)
            o_ref[:, c * _LANE:(c + 1) * _LANE] = q * jax.lax.rsqrt(q) - dr

    @pl.when(jnp.logical_not(r_is_half))
    def _():
        for c in range(n_chunks):
            q = chunk_q(c)
            o_ref[:, c * _LANE:(c + 1) * _LANE] = (
                jnp.exp2(c_r * jnp.log(q)) - dr)


def kernel(x, alpha, r, delta):
    nb, T = x.shape
    af = jnp.asarray(alpha, jnp.float32)
    rf = jnp.asarray(r, jnp.float32)
    df = jnp.asarray(delta, jnp.float32)
    scal = jnp.stack([rf, df, df ** rf, -af * _LOG2E, rf * _LOG2E])
    w = jnp.asarray(_W)
    return pl.pallas_call(
        _pcen_body,
        out_shape=jax.ShapeDtypeStruct((nb, T), jnp.float32),
        grid=(nb // _BB,),
        in_specs=[
            pl.BlockSpec(memory_space=pltpu.SMEM),
            pl.BlockSpec((2 * _LANE, _LANE), lambda i: (0, 0)),
            pl.BlockSpec((_BB, T), lambda i: (i, 0)),
        ],
        out_specs=pl.BlockSpec((_BB, T), lambda i: (i, 0)),
        compiler_params=pltpu.CompilerParams(
            dimension_semantics=("parallel",),
        ),
        name="pcen",
    )(scal, w, x)
